# Initial kernel scaffold; baseline (speedup 1.0000x reference)
import jax
import jax.numpy as jnp
from jax.experimental import pallas as pl


def kernel(x, edge_index, batch, W1, b1, W2, b2, W3, b3, fW1, fb1, fW2, fb2):
    raise NotImplementedError("write your pallas kernel here")


# speedup vs baseline: 19.3175x; 19.3175x over previous
import functools

import jax
import jax.numpy as jnp
from jax import lax
from jax.experimental import pallas as pl
from jax.experimental.pallas import tpu as pltpu
from jax.experimental.pallas import tpu_sc as plsc

N = 10000
E = 320000
D = 128
G = 64
T = 10
NC, NS, L = 2, 16, 16
NW = NC * NS
CH = 128
CPW = -(-E // (NW * CH))
EPAD = NW * CPW * CH
NPAD = N + 112
RPT = NPAD // NS
R = 2528
GRID = NPAD // R
F32 = jnp.float32
HIGH = lax.Precision.HIGHEST


def _mesh():
    return plsc.VectorSubcoreMesh(
        core_axis_name="c", subcore_axis_name="s",
        num_cores=NC, num_subcores=NS)


def _zero_rows(rows_v, nrows, ncols):
    z16 = jnp.zeros((L,), F32)

    def zrow(r, _):
        for j in range(ncols // L):
            rows_v[r, pl.ds(j * L, L)] = z16
        return 0

    lax.fori_loop(0, nrows, zrow, 0)


def _zero_acc_share(zsrc, acc, s):
    base = s * RPT
    off = 0
    while off < RPT:
        sz = min(CH, RPT - off)
        pltpu.sync_copy(zsrc.at[pl.ds(0, sz)], acc.at[pl.ds(base + off, sz)])
        off += sz


def _make_propagate(F):
    @functools.partial(
        pl.kernel,
        out_type=jax.ShapeDtypeStruct((NC, NPAD, F), F32),
        mesh=_mesh(),
        scratch_types=[
            pltpu.VMEM((CPW, CH), jnp.int32),
            pltpu.VMEM((CPW, CH), jnp.int32),
            pltpu.VMEM((CH, F), F32),
            pltpu.VMEM_SHARED((NPAD, F), F32),
            pltpu.SemaphoreType.DMA,
        ],
        compiler_params=pltpu.CompilerParams(use_tc_tiling_on_sc=False),
    )
    def prop(src_hbm, dst_hbm, h_hbm, out_hbm, src_v, dst_v, rows_v, acc, sem):
        c = lax.axis_index("c")
        s = lax.axis_index("s")
        wid = c * NS + s
        _zero_rows(rows_v, CH, F)
        _zero_acc_share(rows_v, acc, s)
        pltpu.sync_copy(src_hbm.at[wid], src_v)
        pltpu.sync_copy(dst_hbm.at[wid], dst_v)
        plsc.subcore_barrier()

        def body(i, _):
            pltpu.async_copy(h_hbm.at[src_v.at[i]], rows_v, sem).wait()
            pltpu.sync_copy(rows_v, acc.at[dst_v.at[i]], add=True)
            return 0

        lax.fori_loop(0, CPW, body, 0)
        plsc.subcore_barrier()
        base = s * RPT
        pltpu.sync_copy(acc.at[pl.ds(base, RPT)],
                        out_hbm.at[c, pl.ds(base, RPT)])

    return prop


def _make_degree():
    F = 16

    @functools.partial(
        pl.kernel,
        out_type=jax.ShapeDtypeStruct((NC, NPAD, F), F32),
        mesh=_mesh(),
        scratch_types=[
            pltpu.VMEM((CPW, CH), jnp.int32),
            pltpu.VMEM((CH, F), F32),
            pltpu.VMEM((CH, F), F32),
            pltpu.VMEM_SHARED((NPAD, F), F32),
        ],
        compiler_params=pltpu.CompilerParams(use_tc_tiling_on_sc=False),
    )
    def deg(dst_hbm, out_hbm, dst_v, ones_v, zeros_v, deg_acc):
        c = lax.axis_index("c")
        s = lax.axis_index("s")
        wid = c * NS + s
        one_row = jnp.where(lax.iota(jnp.int32, L) == 0, 1.0, 0.0)
        z16 = jnp.zeros((L,), F32)

        def init(r, _):
            ones_v[r, pl.ds(0, L)] = one_row
            zeros_v[r, pl.ds(0, L)] = z16
            return 0

        lax.fori_loop(0, CH, init, 0)
        _zero_acc_share(zeros_v, deg_acc, s)
        pltpu.sync_copy(dst_hbm.at[wid], dst_v)
        plsc.subcore_barrier()

        def body(i, _):
            pltpu.sync_copy(ones_v, deg_acc.at[dst_v.at[i]], add=True)
            return 0

        lax.fori_loop(0, CPW, body, 0)
        plsc.subcore_barrier()
        base = s * RPT
        pltpu.sync_copy(deg_acc.at[pl.ds(base, RPT)],
                        out_hbm.at[c, pl.ds(base, RPT)])

    return deg


def _tc1_body(d_ref, x_ref, w1_ref, dinv_ref, s1_ref):
    degsum = d_ref[0][:, 0:1] + d_ref[1][:, 0:1] + 1.0
    dinv = lax.rsqrt(degsum)
    dinv_ref[...] = dinv
    h = jnp.dot(x_ref[...], w1_ref[...],
                preferred_element_type=F32, precision=HIGH)
    s1_ref[...] = h * dinv


def _tc2_body(a_ref, s1_ref, dinv_ref, b1_ref, s2_ref):
    dv = dinv_ref[...]
    y1 = jnp.maximum(
        (a_ref[0] + a_ref[1] + s1_ref[...]) * dv + b1_ref[0:1, :], 0.0)
    s2_ref[...] = y1 * dv


def _tc3_body(a_ref, s2_ref, dinv_ref, w2_ref, b2_ref, s3_ref):
    dv = dinv_ref[...]
    q = (a_ref[0] + a_ref[1] + s2_ref[...]) * dv
    y2 = jnp.maximum(
        jnp.dot(q, w2_ref[...], preferred_element_type=F32, precision=HIGH)
        + b2_ref[0:1, :], 0.0)
    s3_ref[...] = y2 * dv


def _tc4_body(a_ref, s3_ref, dinv_ref, w3_ref, b3_ref, bat_ref,
              fw1_ref, fb1_ref, fw2_ref, fb2_ref, out_ref, pooled, cnt):
    i = pl.program_id(0)
    dv = dinv_ref[...]
    q = (a_ref[0] + a_ref[1] + s3_ref[...]) * dv
    y3 = jnp.dot(q, w3_ref[...],
                 preferred_element_type=F32, precision=HIGH) + b3_ref[0:1, :]
    gid = lax.broadcasted_iota(jnp.int32, (R, G), 1)
    m = (bat_ref[...] == gid).astype(F32)
    pm = lax.dot_general(m, y3, (((0,), (0,)), ((), ())),
                         preferred_element_type=F32, precision=HIGH)
    ones_col = jnp.ones((R, 1), F32)
    pc = lax.dot_general(m, ones_col, (((0,), (0,)), ((), ())),
                         preferred_element_type=F32, precision=HIGH)

    @pl.when(i == 0)
    def _():
        pooled[...] = pm
        cnt[...] = pc

    @pl.when(i != 0)
    def _():
        pooled[...] = pooled[...] + pm
        cnt[...] = cnt[...] + pc

    @pl.when(i == GRID - 1)
    def _():
        pmean = pooled[...] / jnp.maximum(cnt[...], 1.0)
        z = jnp.maximum(
            jnp.dot(pmean, fw1_ref[...],
                    preferred_element_type=F32, precision=HIGH)
            + fb1_ref[0:1, :], 0.0)
        out_ref[...] = jnp.dot(z, fw2_ref[...],
                               preferred_element_type=F32, precision=HIGH
                               ) + fb2_ref[0:1, :]


def _row_spec(f):
    return pl.BlockSpec((R, f), lambda i: (i, 0))


def _pair_spec(f):
    return pl.BlockSpec((NC, R, f), lambda i: (0, i, 0))


def _full_spec(r, f):
    return pl.BlockSpec((r, f), lambda i: (0, 0))


_DEG = _make_degree()
_PROP32 = _make_propagate(32)
_PROP64 = _make_propagate(64)


def kernel(x, edge_index, batch, W1, b1, W2, b2, W3, b3, fW1, fb1, fW2, fb2):
    src = edge_index[0]
    dst = edge_index[1]
    pad = EPAD - E
    src_p = jnp.concatenate(
        [src, jnp.zeros((pad,), jnp.int32)]).reshape(NW, CPW, CH)
    dst_p = jnp.concatenate(
        [dst, jnp.full((pad,), N, jnp.int32)]).reshape(NW, CPW, CH)
    xp = jnp.pad(x, ((0, NPAD - N), (0, 0)))
    bat = jnp.pad(batch, (0, NPAD - N), constant_values=G).reshape(NPAD, 1)
    b1e = jnp.broadcast_to(b1, (8, 32))
    b2e = jnp.broadcast_to(b2, (8, 64))
    b3e = jnp.broadcast_to(b3, (8, 128))
    fb1e = jnp.broadcast_to(fb1, (8, 64))
    fb2e = jnp.broadcast_to(fb2, (8, T))

    degp = _DEG(dst_p)

    dinv, s1 = pl.pallas_call(
        _tc1_body,
        grid=(GRID,),
        in_specs=[_pair_spec(16), _row_spec(D), _full_spec(D, 32)],
        out_specs=[_row_spec(1), _row_spec(32)],
        out_shape=[jax.ShapeDtypeStruct((NPAD, 1), F32),
                   jax.ShapeDtypeStruct((NPAD, 32), F32)],
    )(degp, xp, W1)

    acc1 = _PROP32(src_p, dst_p, s1)

    s2 = pl.pallas_call(
        _tc2_body,
        grid=(GRID,),
        in_specs=[_pair_spec(32), _row_spec(32), _row_spec(1),
                  _full_spec(8, 32)],
        out_specs=_row_spec(32),
        out_shape=jax.ShapeDtypeStruct((NPAD, 32), F32),
    )(acc1, s1, dinv, b1e)

    acc2 = _PROP32(src_p, dst_p, s2)

    s3 = pl.pallas_call(
        _tc3_body,
        grid=(GRID,),
        in_specs=[_pair_spec(32), _row_spec(32), _row_spec(1),
                  _full_spec(32, 64), _full_spec(8, 64)],
        out_specs=_row_spec(64),
        out_shape=jax.ShapeDtypeStruct((NPAD, 64), F32),
    )(acc2, s2, dinv, W2, b2e)

    acc3 = _PROP64(src_p, dst_p, s3)

    out = pl.pallas_call(
        _tc4_body,
        grid=(GRID,),
        in_specs=[_pair_spec(64), _row_spec(64), _row_spec(1),
                  _full_spec(64, D), _full_spec(8, D),
                  pl.BlockSpec((R, 1), lambda i: (i, 0)),
                  _full_spec(D, 64), _full_spec(8, 64),
                  _full_spec(64, T), _full_spec(8, T)],
        out_specs=pl.BlockSpec((G, T), lambda i: (0, 0)),
        out_shape=jax.ShapeDtypeStruct((G, T), F32),
        scratch_shapes=[pltpu.VMEM((G, D), F32), pltpu.VMEM((G, 1), F32)],
    )(acc3, s3, dinv, W3, b3e, bat, fW1, fb1e, fW2, fb2e)

    return out
